# baseline (speedup 1.0000x reference)
import functools
import jax
import jax.numpy as jnp
from jax import lax
from jax.experimental import pallas as pl
from jax.experimental.pallas import tpu as pltpu
from jax.experimental.pallas import tpu_sc as plsc

B = 16384
K = 20
D = 64
NW = 32
BW = B // NW
C = 32
NSTEP = BW // C
IDXROW = 128
XROWS_PER_CHUNK = (C * K) // IDXROW


def _sc_kernel(cl_hbm, xl_hbm, ctab_hbm, xtab_hbm, out_hbm,
               cidx, xidx, crow, xrow, obuf, sbuf, sem):
    nc = 2
    wid = lax.axis_index("s") * nc + lax.axis_index("c")

    iota = lax.iota(jnp.int32, 16)
    hi_mask = iota < 4

    pltpu.sync_copy(cl_hbm.at[wid], cidx)
    pltpu.sync_copy(xl_hbm.at[wid], xidx)

    def step(s, _):
        handles = []
        handles.append(pltpu.async_copy(
            ctab_hbm.at[cidx.at[pl.ds(s * C, C)]], crow, sem))
        for j in range(XROWS_PER_CHUNK):
            handles.append(pltpu.async_copy(
                xtab_hbm.at[xidx.at[s * XROWS_PER_CHUNK + j]],
                xrow.at[pl.ds(j * IDXROW, IDXROW)], sem))
        for h in handles:
            h.wait()

        def per_center(b, _):
            c0 = crow[b, pl.ds(0, 16)]
            c1 = crow[b, pl.ds(16, 16)]
            c2 = crow[b, pl.ds(32, 16)]
            c3 = crow[b, pl.ds(48, 16)]
            for k in range(K):
                r = b * K + k
                p = c0 * xrow[r, pl.ds(0, 16)]
                p = p + c1 * xrow[r, pl.ds(16, 16)]
                p = p + c2 * xrow[r, pl.ds(32, 16)]
                p = p + c3 * xrow[r, pl.ds(48, 16)]
                plsc.store_scatter(sbuf, [iota * 32 + k], p)
            s_lo = sbuf[pl.ds(0, 16)]
            s_hi = sbuf[pl.ds(16, 16)]
            for l in range(1, 16):
                s_lo = s_lo + sbuf[pl.ds(l * 32, 16)]
                s_hi = s_hi + sbuf[pl.ds(l * 32 + 16, 16)]
            plsc.store_scatter(obuf, [b * K + iota], s_lo)
            plsc.store_scatter(obuf, [jnp.minimum(b * K + 16 + iota, C * K - 1)],
                               s_hi, mask=hi_mask)
            return 0

        lax.fori_loop(0, C, per_center, 0)
        pltpu.sync_copy(obuf, out_hbm.at[pl.ds(wid * BW * K + s * C * K, C * K)])
        return 0

    lax.fori_loop(0, NSTEP, step, 0)


@jax.jit
def kernel(center_labels, context_labels, center_table, context_table):
    mesh = plsc.VectorSubcoreMesh(core_axis_name="c", subcore_axis_name="s")
    k = functools.partial(
        pl.kernel,
        out_type=jax.ShapeDtypeStruct((B * K,), jnp.float32),
        mesh=mesh,
        compiler_params=pltpu.CompilerParams(needs_layout_passes=False,
                                             use_tc_tiling_on_sc=False),
        scratch_types=[
            pltpu.VMEM((BW,), jnp.int32),
            pltpu.VMEM((BW * K // IDXROW, IDXROW), jnp.int32),
            pltpu.VMEM((C, D), jnp.float32),
            pltpu.VMEM((C * K, D), jnp.float32),
            pltpu.VMEM((C * K,), jnp.float32),
            pltpu.VMEM((16 * 32,), jnp.float32),
            pltpu.SemaphoreType.DMA,
        ],
    )(_sc_kernel)
    out = k(center_labels.reshape(NW, BW),
            context_labels.reshape(NW, BW * K // IDXROW, IDXROW),
            center_table, context_table)
    return out.reshape(B, K)


# speedup vs baseline: 1.6696x; 1.6696x over previous
import functools
import jax
import jax.numpy as jnp
from jax import lax
from jax.experimental import pallas as pl
from jax.experimental.pallas import tpu as pltpu
from jax.experimental.pallas import tpu_sc as plsc

VOCAB = 1000000
B = 16384
K = 20
D = 64
NW = 32
BW = B // NW
C = 32
NSTEP = BW // C
IDXROW = 128
XROWS_PER_CHUNK = (C * K) // IDXROW
TBLK = 4096


def _pack_kernel(ct_ref, xt_ref, out_ref):
    out_ref[:, 0:D] = ct_ref[...].T
    out_ref[:, D:2 * D] = xt_ref[...].T


def _sc_kernel(cl_hbm, xl_hbm, tab_hbm, out_hbm,
               cidx, xidx, crow, xrow, obuf, sbuf, sem):
    nc = 2
    wid = lax.axis_index("s") * nc + lax.axis_index("c")

    iota = lax.iota(jnp.int32, 16)
    hi_mask = iota < 4

    pltpu.sync_copy(cl_hbm.at[wid], cidx)
    pltpu.sync_copy(xl_hbm.at[wid], xidx)

    def step(s, _):
        handles = []
        handles.append(pltpu.async_copy(
            tab_hbm.at[cidx.at[pl.ds(s * C, C)]], crow, sem))
        for j in range(XROWS_PER_CHUNK):
            handles.append(pltpu.async_copy(
                tab_hbm.at[xidx.at[s * XROWS_PER_CHUNK + j]],
                xrow.at[pl.ds(j * IDXROW, IDXROW)], sem))
        for h in handles:
            h.wait()

        def per_center(b, _):
            c0 = crow[b, pl.ds(0, 16)]
            c1 = crow[b, pl.ds(16, 16)]
            c2 = crow[b, pl.ds(32, 16)]
            c3 = crow[b, pl.ds(48, 16)]
            for k in range(K):
                r = b * K + k
                p = c0 * xrow[r, pl.ds(D, 16)]
                p = p + c1 * xrow[r, pl.ds(D + 16, 16)]
                p = p + c2 * xrow[r, pl.ds(D + 32, 16)]
                p = p + c3 * xrow[r, pl.ds(D + 48, 16)]
                plsc.store_scatter(sbuf, [iota * 32 + k], p)
            s_lo = sbuf[pl.ds(0, 16)]
            s_hi = sbuf[pl.ds(16, 16)]
            for l in range(1, 16):
                s_lo = s_lo + sbuf[pl.ds(l * 32, 16)]
                s_hi = s_hi + sbuf[pl.ds(l * 32 + 16, 16)]
            plsc.store_scatter(obuf, [b * K + iota], s_lo)
            plsc.store_scatter(obuf, [jnp.minimum(b * K + 16 + iota, C * K - 1)],
                               s_hi, mask=hi_mask)
            return 0

        lax.fori_loop(0, C, per_center, 0)
        pltpu.sync_copy(obuf, out_hbm.at[pl.ds(wid * BW * K + s * C * K, C * K)])
        return 0

    lax.fori_loop(0, NSTEP, step, 0)


@jax.jit
def kernel(center_labels, context_labels, center_table, context_table):
    nblk = (VOCAB + TBLK - 1) // TBLK
    packed = pl.pallas_call(
        _pack_kernel,
        grid=(nblk,),
        in_specs=[
            pl.BlockSpec((D, TBLK), lambda i: (0, i)),
            pl.BlockSpec((D, TBLK), lambda i: (0, i)),
        ],
        out_specs=pl.BlockSpec((TBLK, 2 * D), lambda i: (i, 0)),
        out_shape=jax.ShapeDtypeStruct((VOCAB, 2 * D), jnp.float32),
    )(center_table.T, context_table.T)

    mesh = plsc.VectorSubcoreMesh(core_axis_name="c", subcore_axis_name="s")
    k = functools.partial(
        pl.kernel,
        out_type=jax.ShapeDtypeStruct((B * K,), jnp.float32),
        mesh=mesh,
        compiler_params=pltpu.CompilerParams(needs_layout_passes=False,
                                             use_tc_tiling_on_sc=True),
        scratch_types=[
            pltpu.VMEM((BW,), jnp.int32),
            pltpu.VMEM((BW * K // IDXROW, IDXROW), jnp.int32),
            pltpu.VMEM((C, 2 * D), jnp.float32),
            pltpu.VMEM((C * K, 2 * D), jnp.float32),
            pltpu.VMEM((C * K,), jnp.float32),
            pltpu.VMEM((16 * 32,), jnp.float32),
            pltpu.SemaphoreType.DMA,
        ],
    )(_sc_kernel)
    out = k(center_labels.reshape(NW, BW),
            context_labels.reshape(NW, BW * K // IDXROW, IDXROW),
            packed)
    return out.reshape(B, K)
